# Initial kernel scaffold; baseline (speedup 1.0000x reference)
import jax
import jax.numpy as jnp
from jax.experimental import pallas as pl


def kernel(idx, targets, table):
    raise NotImplementedError("write your pallas kernel here")


# speedup vs baseline: 1.3665x; 1.3665x over previous
import functools

import jax
import jax.numpy as jnp
from jax import lax
from jax.experimental import pallas as pl
from jax.experimental.pallas import tpu as pltpu
from jax.experimental.pallas import tpu_sc as plsc

_VOCAB = 1000
_NC = 2
_NS = 16
_NW = _NC * _NS
_LANES = 16
_CH = 32


def _lse_body(t_ref, o_ref):
    x = t_ref[...]
    m = jnp.max(x, axis=1)
    s = jnp.sum(jnp.exp(x - m[:, None]), axis=1)
    o_ref[...] = m + jnp.log(s)


def _row_lse(table):
    return pl.pallas_call(
        _lse_body,
        out_shape=jax.ShapeDtypeStruct((table.shape[0],), jnp.float32),
    )(table)


def _make_sc_kernel(n_tok):
    per_w = n_tok // _NW
    n_chunks = per_w // _CH
    mesh = plsc.VectorSubcoreMesh(core_axis_name="c", subcore_axis_name="s")

    @functools.partial(
        pl.kernel,
        out_type=(
            jax.ShapeDtypeStruct((n_tok, _VOCAB), jnp.float32),
            jax.ShapeDtypeStruct((_NW, _LANES), jnp.float32),
        ),
        mesh=mesh,
        compiler_params=pltpu.CompilerParams(
            use_tc_tiling_on_sc=False, needs_layout_passes=False),
        scratch_types=[
            pltpu.VMEM((per_w,), jnp.int32),
            pltpu.VMEM((per_w,), jnp.int32),
            pltpu.VMEM((_VOCAB,), jnp.float32),
            pltpu.VMEM((_CH, _VOCAB), jnp.float32),
            pltpu.VMEM((_LANES,), jnp.float32),
            pltpu.SemaphoreType.DMA,
        ],
    )
    def sc_kernel(table_hbm, idx_hbm, tgt_hbm, lse_hbm, logits_hbm, part_hbm,
                  idx_v, tgt_v, lse_v, buf, acc_v, gsem):
        wid = lax.axis_index("s") * _NC + lax.axis_index("c")
        base = wid * per_w
        pltpu.sync_copy(idx_hbm.at[pl.ds(base, per_w)], idx_v)
        pltpu.sync_copy(tgt_hbm.at[pl.ds(base, per_w)], tgt_v)
        pltpu.sync_copy(lse_hbm, lse_v)
        acc_v[...] = jnp.zeros((_LANES,), jnp.float32)

        def body(c, carry):
            cb = c * _CH
            pltpu.async_copy(
                table_hbm.at[idx_v.at[pl.ds(cb, _CH)]], buf, gsem).wait()
            for g in range(_CH // _LANES):
                off = cb + g * _LANES
                rid = lax.broadcasted_iota(jnp.int32, (_LANES,), 0) + g * _LANES
                tg = tgt_v[pl.ds(off, _LANES)]
                ig = idx_v[pl.ds(off, _LANES)]
                picked = plsc.load_gather(buf, [rid, tg])
                lseg = plsc.load_gather(lse_v, [ig])
                acc_v[...] = acc_v[...] + (lseg - picked)
            pltpu.sync_copy(buf, logits_hbm.at[pl.ds(base + cb, _CH)])
            return carry

        lax.fori_loop(0, n_chunks, body, 0)
        pltpu.sync_copy(acc_v, part_hbm.at[wid])

    return sc_kernel


def kernel(idx, targets, table):
    b, l = idx.shape
    n_tok = b * l
    idx_f = idx.reshape(n_tok).astype(jnp.int32)
    tgt_f = targets.reshape(n_tok).astype(jnp.int32)
    lse = _row_lse(table)
    logits_flat, partials = _make_sc_kernel(n_tok)(table, idx_f, tgt_f, lse)
    loss = jnp.sum(partials) / n_tok
    return (logits_flat.reshape(b, l, _VOCAB), loss)


# speedup vs baseline: 1.4176x; 1.0373x over previous
import functools

import jax
import jax.numpy as jnp
from jax import lax
from jax.experimental import pallas as pl
from jax.experimental.pallas import tpu as pltpu
from jax.experimental.pallas import tpu_sc as plsc

_VOCAB = 1000
_NC = 2
_NS = 16
_NW = _NC * _NS
_LANES = 16
_CH = 32


def _lse_body(t_ref, o_ref):
    x = t_ref[...]
    m = jnp.max(x, axis=1)
    s = jnp.sum(jnp.exp(x - m[:, None]), axis=1)
    o_ref[...] = m + jnp.log(s)


def _row_lse(table):
    return pl.pallas_call(
        _lse_body,
        out_shape=jax.ShapeDtypeStruct((table.shape[0],), jnp.float32),
    )(table)


def _make_sc_kernel(n_tok):
    per_w = n_tok // _NW
    n_chunks = per_w // _CH
    n_pairs = n_chunks // 2
    mesh = plsc.VectorSubcoreMesh(core_axis_name="c", subcore_axis_name="s")

    @functools.partial(
        pl.kernel,
        out_type=(
            jax.ShapeDtypeStruct((n_tok, _VOCAB), jnp.float32),
            jax.ShapeDtypeStruct((_NW, _LANES), jnp.float32),
        ),
        mesh=mesh,
        compiler_params=pltpu.CompilerParams(
            use_tc_tiling_on_sc=False, needs_layout_passes=False),
        scratch_types=[
            pltpu.VMEM((per_w,), jnp.int32),
            pltpu.VMEM((per_w,), jnp.int32),
            pltpu.VMEM((_VOCAB,), jnp.float32),
            pltpu.VMEM((_CH, _VOCAB), jnp.float32),
            pltpu.VMEM((_CH, _VOCAB), jnp.float32),
            pltpu.VMEM((_LANES,), jnp.float32),
            pltpu.SemaphoreType.DMA,
            pltpu.SemaphoreType.DMA,
            pltpu.SemaphoreType.DMA,
            pltpu.SemaphoreType.DMA,
        ],
    )
    def sc_kernel(table_hbm, idx_hbm, tgt_hbm, lse_hbm, logits_hbm, part_hbm,
                  idx_v, tgt_v, lse_v, buf_a, buf_b, acc_v,
                  gsem_a, gsem_b, ssem_a, ssem_b):
        wid = lax.axis_index("s") * _NC + lax.axis_index("c")
        base = wid * per_w
        pltpu.sync_copy(idx_hbm.at[pl.ds(base, per_w)], idx_v)
        pltpu.sync_copy(tgt_hbm.at[pl.ds(base, per_w)], tgt_v)
        pltpu.sync_copy(lse_hbm, lse_v)
        acc_v[...] = jnp.zeros((_LANES,), jnp.float32)

        def gstart(chunk, buf, gsem):
            pltpu.make_async_copy(
                table_hbm.at[idx_v.at[pl.ds(chunk * _CH, _CH)]], buf, gsem
            ).start()

        def gwait(buf, gsem):
            pltpu.make_async_copy(
                table_hbm.at[idx_v.at[pl.ds(0, _CH)]], buf, gsem).wait()

        def sstart(chunk, buf, ssem):
            pltpu.make_async_copy(
                buf, logits_hbm.at[pl.ds(base + chunk * _CH, _CH)], ssem
            ).start()

        def swait(buf, ssem):
            pltpu.make_async_copy(
                buf, logits_hbm.at[pl.ds(base, _CH)], ssem).wait()

        def loss(chunk, buf):
            cb = chunk * _CH
            for g in range(_CH // _LANES):
                off = cb + g * _LANES
                rid = (lax.broadcasted_iota(jnp.int32, (_LANES,), 0)
                       + g * _LANES)
                tg = tgt_v[pl.ds(off, _LANES)]
                ig = idx_v[pl.ds(off, _LANES)]
                picked = plsc.load_gather(buf, [rid, tg])
                lseg = plsc.load_gather(lse_v, [ig])
                acc_v[...] = acc_v[...] + (lseg - picked)

        gstart(0, buf_a, gsem_a)
        gstart(1, buf_b, gsem_b)

        def pair_body(p, carry):
            k = 2 * p
            gwait(buf_a, gsem_a)
            loss(k, buf_a)
            sstart(k, buf_a, ssem_a)
            gwait(buf_b, gsem_b)
            loss(k + 1, buf_b)
            sstart(k + 1, buf_b, ssem_b)
            swait(buf_a, ssem_a)
            gstart(k + 2, buf_a, gsem_a)
            swait(buf_b, ssem_b)
            gstart(k + 3, buf_b, gsem_b)
            return carry

        lax.fori_loop(0, n_pairs - 1, pair_body, 0)

        k = n_chunks - 2
        gwait(buf_a, gsem_a)
        loss(k, buf_a)
        sstart(k, buf_a, ssem_a)
        gwait(buf_b, gsem_b)
        loss(k + 1, buf_b)
        sstart(k + 1, buf_b, ssem_b)
        swait(buf_a, ssem_a)
        swait(buf_b, ssem_b)
        pltpu.sync_copy(acc_v, part_hbm.at[wid])

    return sc_kernel


def kernel(idx, targets, table):
    b, l = idx.shape
    n_tok = b * l
    idx_f = idx.reshape(n_tok).astype(jnp.int32)
    tgt_f = targets.reshape(n_tok).astype(jnp.int32)
    lse = _row_lse(table)
    logits_flat, partials = _make_sc_kernel(n_tok)(table, idx_f, tgt_f, lse)
    loss = jnp.sum(partials) / n_tok
    return (logits_flat.reshape(b, l, _VOCAB), loss)
